# Initial kernel scaffold; baseline (speedup 1.0000x reference)
import jax
import jax.numpy as jnp
from jax.experimental import pallas as pl


def kernel(x, edge_index, W1, b1, W2, b2):
    raise NotImplementedError("write your pallas kernel here")


# speedup vs baseline: 28.2065x; 28.2065x over previous
import functools

import jax
import jax.numpy as jnp
from jax import lax
from jax.experimental import pallas as pl
from jax.experimental.pallas import tpu as pltpu
from jax.experimental.pallas import tpu_sc as plsc

N = 10000
E = 320000
F_IN = 128
N_HID = 16
N_CLS = 40

NC = 2
NS = 16
NW = NC * NS
CK = 128
NCH = 80
E_PAD = NW * NCH * CK
R_PAD = 10240
RPS = R_PAD // NS
NBUF = 4

_mesh = plsc.VectorSubcoreMesh(core_axis_name="c", subcore_axis_name="s")
_sc_params = pltpu.CompilerParams(use_tc_tiling_on_sc=False)


def _deg_body(col_hbm, zer_hbm, out_hbm, col_v, ones_v, acc, sem):
    c = lax.axis_index("c")
    s = lax.axis_index("s")
    wid = s * NC + c
    sl = pl.ds(s * RPS, RPS)

    @pl.loop(0, CK, step=16)
    def _(i):
        ones_v[pl.ds(i, 16)] = jnp.full((16,), 1.0, jnp.float32)

    pltpu.sync_copy(zer_hbm.at[sl], acc.at[sl])
    pltpu.sync_copy(col_hbm.at[wid], col_v)
    plsc.subcore_barrier()

    @pl.loop(0, NCH, step=20)
    def _(j):
        for b in range(20):
            pltpu.async_copy(ones_v, acc.at[col_v.at[j + b]], sem, add=True)
        for b in range(20):
            pltpu.make_async_copy(ones_v, acc.at[col_v.at[j + b]], sem).wait()

    plsc.subcore_barrier()
    pltpu.sync_copy(acc.at[sl], out_hbm.at[c].at[sl])


def _scatter_body(D, g_hbm, zer_hbm, row_hbm, col_hbm, out_hbm,
                  row_v, col_v, bufs, acc, gsem):
    c = lax.axis_index("c")
    s = lax.axis_index("s")
    wid = s * NC + c
    sl = pl.ds(s * RPS, RPS)

    @pl.when(c == 0)
    def _():
        pltpu.sync_copy(g_hbm.at[sl], acc.at[sl])

    @pl.when(c != 0)
    def _():
        pltpu.sync_copy(zer_hbm.at[sl], acc.at[sl])

    pltpu.sync_copy(row_hbm.at[wid], row_v)
    pltpu.sync_copy(col_hbm.at[wid], col_v)
    plsc.subcore_barrier()

    def start_gather(j, b):
        pltpu.async_copy(g_hbm.at[row_v.at[j]], bufs.at[b], gsem.at[b])

    def wait_gather(j, b):
        pltpu.make_async_copy(g_hbm.at[row_v.at[j]], bufs.at[b],
                              gsem.at[b]).wait()

    for b in range(NBUF):
        start_gather(b, b)

    @pl.loop(0, NCH - NBUF, step=NBUF)
    def _(j):
        for b in range(NBUF):
            wait_gather(j + b, b)
            pltpu.sync_copy(bufs.at[b], acc.at[col_v.at[j + b]], add=True)
            start_gather(j + b + NBUF, b)

    for b in range(NBUF):
        jj = NCH - NBUF + b
        wait_gather(jj, b)
        pltpu.sync_copy(bufs.at[b], acc.at[col_v.at[jj]], add=True)

    plsc.subcore_barrier()
    pltpu.sync_copy(acc.at[sl], out_hbm.at[c].at[sl])


def _deg_call(colp, zer1):
    return pl.kernel(
        _deg_body,
        out_type=jax.ShapeDtypeStruct((NC, R_PAD), jnp.float32),
        mesh=_mesh,
        scratch_types=[
            pltpu.VMEM((NCH, CK), jnp.int32),
            pltpu.VMEM((CK,), jnp.float32),
            pltpu.VMEM_SHARED((R_PAD,), jnp.float32),
            pltpu.SemaphoreType.DMA,
        ],
        compiler_params=_sc_params,
    )(colp, zer1)


def _scatter_call(D, g, zer, rowp, colp):
    return pl.kernel(
        functools.partial(_scatter_body, D),
        out_type=jax.ShapeDtypeStruct((NC, R_PAD, D), jnp.float32),
        mesh=_mesh,
        scratch_types=[
            pltpu.VMEM((NCH, CK), jnp.int32),
            pltpu.VMEM((NCH, CK), jnp.int32),
            pltpu.VMEM((NBUF, CK, D), jnp.float32),
            pltpu.VMEM_SHARED((R_PAD, D), jnp.float32),
            pltpu.SemaphoreType.DMA((NBUF,)),
        ],
        compiler_params=_sc_params,
    )(g, zer, rowp, colp)


_BLK = 1024
_OBLK = 1000


def _mm1_body(x_ref, w_ref, o_ref):
    o_ref[...] = jnp.dot(x_ref[...], w_ref[...],
                         preferred_element_type=jnp.float32)


def _scale_body(p0_ref, p1_ref, h_ref, g_ref, d_ref):
    deg = p0_ref[...] + p1_ref[...] + 1.0
    d = lax.rsqrt(deg)
    d_ref[...] = d
    g_ref[...] = h_ref[...] * d


def _mid_body(a0_ref, a1_ref, d_ref, b1_ref, w2_ref, g2_ref):
    d = d_ref[...]
    o1 = jnp.maximum((a0_ref[...] + a1_ref[...]) * d + b1_ref[...], 0.0)
    h2 = jnp.dot(o1, w2_ref[...], preferred_element_type=jnp.float32)
    g2_ref[...] = h2 * d


def _fin_body(a0_ref, a1_ref, d_ref, b2_ref, o_ref):
    o = (a0_ref[...] + a1_ref[...]) * d_ref[...] + b2_ref[...]
    m = jnp.max(o, axis=1, keepdims=True)
    e = jnp.exp(o - m)
    lse = jnp.log(jnp.sum(e, axis=1, keepdims=True))
    o_ref[...] = o - m - lse


def _mm1(x_p, W1):
    return pl.pallas_call(
        _mm1_body,
        grid=(R_PAD // _BLK,),
        in_specs=[pl.BlockSpec((_BLK, F_IN), lambda i: (i, 0)),
                  pl.BlockSpec((F_IN, N_HID), lambda i: (0, 0))],
        out_specs=pl.BlockSpec((_BLK, N_HID), lambda i: (i, 0)),
        out_shape=jax.ShapeDtypeStruct((R_PAD, N_HID), jnp.float32),
    )(x_p, W1)


def _scale(p0, p1, h):
    return pl.pallas_call(
        _scale_body,
        grid=(R_PAD // _BLK,),
        in_specs=[pl.BlockSpec((_BLK, 1), lambda i: (i, 0)),
                  pl.BlockSpec((_BLK, 1), lambda i: (i, 0)),
                  pl.BlockSpec((_BLK, N_HID), lambda i: (i, 0))],
        out_specs=[pl.BlockSpec((_BLK, N_HID), lambda i: (i, 0)),
                   pl.BlockSpec((_BLK, 1), lambda i: (i, 0))],
        out_shape=[jax.ShapeDtypeStruct((R_PAD, N_HID), jnp.float32),
                   jax.ShapeDtypeStruct((R_PAD, 1), jnp.float32)],
    )(p0, p1, h)


def _mid(a0, a1, d, b1r, W2):
    return pl.pallas_call(
        _mid_body,
        grid=(R_PAD // _BLK,),
        in_specs=[pl.BlockSpec((_BLK, N_HID), lambda i: (i, 0)),
                  pl.BlockSpec((_BLK, N_HID), lambda i: (i, 0)),
                  pl.BlockSpec((_BLK, 1), lambda i: (i, 0)),
                  pl.BlockSpec((1, N_HID), lambda i: (0, 0)),
                  pl.BlockSpec((N_HID, N_CLS), lambda i: (0, 0))],
        out_specs=pl.BlockSpec((_BLK, N_CLS), lambda i: (i, 0)),
        out_shape=jax.ShapeDtypeStruct((R_PAD, N_CLS), jnp.float32),
    )(a0, a1, d, b1r, W2)


def _fin(a0, a1, d, b2r):
    return pl.pallas_call(
        _fin_body,
        grid=(N // _OBLK,),
        in_specs=[pl.BlockSpec((_OBLK, N_CLS), lambda i: (i, 0)),
                  pl.BlockSpec((_OBLK, N_CLS), lambda i: (i, 0)),
                  pl.BlockSpec((_OBLK, 1), lambda i: (i, 0)),
                  pl.BlockSpec((1, N_CLS), lambda i: (0, 0))],
        out_specs=pl.BlockSpec((_OBLK, N_CLS), lambda i: (i, 0)),
        out_shape=jax.ShapeDtypeStruct((N, N_CLS), jnp.float32),
    )(a0, a1, d, b2r)


def kernel(x, edge_index, W1, b1, W2, b2):
    row = edge_index[0]
    col = edge_index[1]
    pad = E_PAD - E
    rowp = jnp.concatenate(
        [row, jnp.zeros((pad,), jnp.int32)]).reshape(NW, NCH, CK)
    colp = jnp.concatenate(
        [col, jnp.full((pad,), N, jnp.int32)]).reshape(NW, NCH, CK)
    x_p = jnp.pad(x, ((0, R_PAD - N), (0, 0)))

    zer1 = jnp.zeros((R_PAD,), jnp.float32)
    zer16 = jnp.zeros((R_PAD, N_HID), jnp.float32)
    zer40 = jnp.zeros((R_PAD, N_CLS), jnp.float32)

    degp = _deg_call(colp, zer1)
    h = _mm1(x_p, W1)
    g1, d = _scale(degp[0].reshape(R_PAD, 1), degp[1].reshape(R_PAD, 1), h)
    a1 = _scatter_call(N_HID, g1, zer16, rowp, colp)
    g2 = _mid(a1[0], a1[1], d, b1.reshape(1, N_HID), W2)
    a2 = _scatter_call(N_CLS, g2, zer40, rowp, colp)
    return _fin(a2[0], a2[1], d, b2.reshape(1, N_CLS))


# speedup vs baseline: 28.2349x; 1.0010x over previous
import functools

import jax
import jax.numpy as jnp
from jax import lax
from jax.experimental import pallas as pl
from jax.experimental.pallas import tpu as pltpu
from jax.experimental.pallas import tpu_sc as plsc

N = 10000
E = 320000
F_IN = 128
N_HID = 16
N_CLS = 40

NC = 2
NS = 16
NW = NC * NS
CK = 128
NCH = 80
E_PAD = NW * NCH * CK
R_PAD = 10240
RPS = R_PAD // NS
NBUF = 8
PF = 4

_mesh = plsc.VectorSubcoreMesh(core_axis_name="c", subcore_axis_name="s")
_sc_params = pltpu.CompilerParams(use_tc_tiling_on_sc=False)


def _deg_body(col_hbm, zer_hbm, out_hbm, col_v, ones_v, acc, sem):
    c = lax.axis_index("c")
    s = lax.axis_index("s")
    wid = s * NC + c
    sl = pl.ds(s * RPS, RPS)

    @pl.loop(0, CK, step=16)
    def _(i):
        ones_v[pl.ds(i, 16)] = jnp.full((16,), 1.0, jnp.float32)

    pltpu.sync_copy(zer_hbm.at[sl], acc.at[sl])
    pltpu.sync_copy(col_hbm.at[wid], col_v)
    plsc.subcore_barrier()

    @pl.loop(0, NCH, step=20)
    def _(j):
        for b in range(20):
            pltpu.async_copy(ones_v, acc.at[col_v.at[j + b]], sem, add=True)
        for b in range(20):
            pltpu.make_async_copy(ones_v, acc.at[col_v.at[j + b]], sem).wait()

    plsc.subcore_barrier()
    pltpu.sync_copy(acc.at[sl], out_hbm.at[c].at[sl])


def _scatter_body(D, g_hbm, zer_hbm, row_hbm, col_hbm, out_hbm,
                  row_v, col_v, bufs, acc, gsem, ssem):
    c = lax.axis_index("c")
    s = lax.axis_index("s")
    wid = s * NC + c
    sl = pl.ds(s * RPS, RPS)

    @pl.when(c == 0)
    def _():
        pltpu.sync_copy(g_hbm.at[sl], acc.at[sl])

    @pl.when(c != 0)
    def _():
        pltpu.sync_copy(zer_hbm.at[sl], acc.at[sl])

    pltpu.sync_copy(row_hbm.at[wid], row_v)
    pltpu.sync_copy(col_hbm.at[wid], col_v)
    plsc.subcore_barrier()

    def start_gather(j, b):
        pltpu.async_copy(g_hbm.at[row_v.at[j]], bufs.at[b], gsem.at[b])

    def wait_gather(j, b):
        pltpu.make_async_copy(g_hbm.at[row_v.at[j]], bufs.at[b],
                              gsem.at[b]).wait()

    def start_scatter(j, b):
        pltpu.async_copy(bufs.at[b], acc.at[col_v.at[j]], ssem.at[b],
                         add=True)

    def wait_scatter(j, b):
        pltpu.make_async_copy(bufs.at[b], acc.at[col_v.at[j]],
                              ssem.at[b]).wait()

    for p in range(PF):
        start_gather(p, p)

    @pl.loop(0, NCH, step=NBUF)
    def _(j0):
        for i in range(NBUF):
            j = j0 + i
            bn = (i + PF) % NBUF
            jn = j + PF

            @pl.when(jn < NCH)
            def _():
                @pl.when(jn >= NBUF)
                def _():
                    wait_scatter(jn - NBUF, bn)
                start_gather(jn, bn)

            wait_gather(j, i)
            start_scatter(j, i)

    for b in range(NBUF):
        wait_scatter(0, b)

    plsc.subcore_barrier()
    pltpu.sync_copy(acc.at[sl], out_hbm.at[c].at[sl])


def _deg_call(colp, zer1):
    return pl.kernel(
        _deg_body,
        out_type=jax.ShapeDtypeStruct((NC, R_PAD), jnp.float32),
        mesh=_mesh,
        scratch_types=[
            pltpu.VMEM((NCH, CK), jnp.int32),
            pltpu.VMEM((CK,), jnp.float32),
            pltpu.VMEM_SHARED((R_PAD,), jnp.float32),
            pltpu.SemaphoreType.DMA,
        ],
        compiler_params=_sc_params,
    )(colp, zer1)


def _scatter_call(D, g, zer, rowp, colp):
    return pl.kernel(
        functools.partial(_scatter_body, D),
        out_type=jax.ShapeDtypeStruct((NC, R_PAD, D), jnp.float32),
        mesh=_mesh,
        scratch_types=[
            pltpu.VMEM((NCH, CK), jnp.int32),
            pltpu.VMEM((NCH, CK), jnp.int32),
            pltpu.VMEM((NBUF, CK, D), jnp.float32),
            pltpu.VMEM_SHARED((R_PAD, D), jnp.float32),
            pltpu.SemaphoreType.DMA((NBUF,)),
            pltpu.SemaphoreType.DMA((NBUF,)),
        ],
        compiler_params=_sc_params,
    )(g, zer, rowp, colp)


_BLK = 1024
_OBLK = 1000


def _mm1_body(x_ref, w_ref, o_ref):
    o_ref[...] = jnp.dot(x_ref[...], w_ref[...],
                         preferred_element_type=jnp.float32)


def _scale_body(p0_ref, p1_ref, h_ref, g_ref, d_ref):
    deg = p0_ref[...] + p1_ref[...] + 1.0
    d = lax.rsqrt(deg)
    d_ref[...] = d
    g_ref[...] = h_ref[...] * d


def _mid_body(a0_ref, a1_ref, d_ref, b1_ref, w2_ref, g2_ref):
    d = d_ref[...]
    o1 = jnp.maximum((a0_ref[...] + a1_ref[...]) * d + b1_ref[...], 0.0)
    h2 = jnp.dot(o1, w2_ref[...], preferred_element_type=jnp.float32)
    g2_ref[...] = h2 * d


def _fin_body(a0_ref, a1_ref, d_ref, b2_ref, o_ref):
    o = (a0_ref[...] + a1_ref[...]) * d_ref[...] + b2_ref[...]
    m = jnp.max(o, axis=1, keepdims=True)
    e = jnp.exp(o - m)
    lse = jnp.log(jnp.sum(e, axis=1, keepdims=True))
    o_ref[...] = o - m - lse


def _mm1(x_p, W1):
    return pl.pallas_call(
        _mm1_body,
        grid=(R_PAD // _BLK,),
        in_specs=[pl.BlockSpec((_BLK, F_IN), lambda i: (i, 0)),
                  pl.BlockSpec((F_IN, N_HID), lambda i: (0, 0))],
        out_specs=pl.BlockSpec((_BLK, N_HID), lambda i: (i, 0)),
        out_shape=jax.ShapeDtypeStruct((R_PAD, N_HID), jnp.float32),
    )(x_p, W1)


def _scale(p0, p1, h):
    return pl.pallas_call(
        _scale_body,
        grid=(R_PAD // _BLK,),
        in_specs=[pl.BlockSpec((_BLK, 1), lambda i: (i, 0)),
                  pl.BlockSpec((_BLK, 1), lambda i: (i, 0)),
                  pl.BlockSpec((_BLK, N_HID), lambda i: (i, 0))],
        out_specs=[pl.BlockSpec((_BLK, N_HID), lambda i: (i, 0)),
                   pl.BlockSpec((_BLK, 1), lambda i: (i, 0))],
        out_shape=[jax.ShapeDtypeStruct((R_PAD, N_HID), jnp.float32),
                   jax.ShapeDtypeStruct((R_PAD, 1), jnp.float32)],
    )(p0, p1, h)


def _mid(a0, a1, d, b1r, W2):
    return pl.pallas_call(
        _mid_body,
        grid=(R_PAD // _BLK,),
        in_specs=[pl.BlockSpec((_BLK, N_HID), lambda i: (i, 0)),
                  pl.BlockSpec((_BLK, N_HID), lambda i: (i, 0)),
                  pl.BlockSpec((_BLK, 1), lambda i: (i, 0)),
                  pl.BlockSpec((1, N_HID), lambda i: (0, 0)),
                  pl.BlockSpec((N_HID, N_CLS), lambda i: (0, 0))],
        out_specs=pl.BlockSpec((_BLK, N_CLS), lambda i: (i, 0)),
        out_shape=jax.ShapeDtypeStruct((R_PAD, N_CLS), jnp.float32),
    )(a0, a1, d, b1r, W2)


def _fin(a0, a1, d, b2r):
    return pl.pallas_call(
        _fin_body,
        grid=(N // _OBLK,),
        in_specs=[pl.BlockSpec((_OBLK, N_CLS), lambda i: (i, 0)),
                  pl.BlockSpec((_OBLK, N_CLS), lambda i: (i, 0)),
                  pl.BlockSpec((_OBLK, 1), lambda i: (i, 0)),
                  pl.BlockSpec((1, N_CLS), lambda i: (0, 0))],
        out_specs=pl.BlockSpec((_OBLK, N_CLS), lambda i: (i, 0)),
        out_shape=jax.ShapeDtypeStruct((N, N_CLS), jnp.float32),
    )(a0, a1, d, b2r)


def kernel(x, edge_index, W1, b1, W2, b2):
    row = edge_index[0]
    col = edge_index[1]
    pad = E_PAD - E
    rowp = jnp.concatenate(
        [row, jnp.zeros((pad,), jnp.int32)]).reshape(NW, NCH, CK)
    colp = jnp.concatenate(
        [col, jnp.full((pad,), N, jnp.int32)]).reshape(NW, NCH, CK)
    x_p = jnp.pad(x, ((0, R_PAD - N), (0, 0)))

    zer1 = jnp.zeros((R_PAD,), jnp.float32)
    zer16 = jnp.zeros((R_PAD, N_HID), jnp.float32)
    zer40 = jnp.zeros((R_PAD, N_CLS), jnp.float32)

    degp = _deg_call(colp, zer1)
    h = _mm1(x_p, W1)
    g1, d = _scale(degp[0].reshape(R_PAD, 1), degp[1].reshape(R_PAD, 1), h)
    a1 = _scatter_call(N_HID, g1, zer16, rowp, colp)
    g2 = _mid(a1[0], a1[1], d, b1.reshape(1, N_HID), W2)
    a2 = _scatter_call(N_CLS, g2, zer40, rowp, colp)
    return _fin(a2[0], a2[1], d, b2.reshape(1, N_CLS))


# speedup vs baseline: 45.2978x; 1.6043x over previous
import functools

import jax
import jax.numpy as jnp
from jax import lax
from jax.experimental import pallas as pl
from jax.experimental.pallas import tpu as pltpu
from jax.experimental.pallas import tpu_sc as plsc

N = 10000
E = 320000
F_IN = 128
N_HID = 16
N_CLS = 40

NC = 2
NS = 16
NW = NC * NS
CK = 128
NCH = 80
E_PAD = NW * NCH * CK
R_PAD = 10240
RPS = R_PAD // NS
NBUF = 8
PF = 4

_mesh = plsc.VectorSubcoreMesh(core_axis_name="c", subcore_axis_name="s")
_sc_params = pltpu.CompilerParams(use_tc_tiling_on_sc=False)


def _deg_body(col_hbm, zer_hbm, out_hbm, col_v, ones_v, acc, sem):
    c = lax.axis_index("c")
    s = lax.axis_index("s")
    wid = s * NC + c
    sl = pl.ds(s * RPS, RPS)

    @pl.loop(0, CK, step=16)
    def _(i):
        ones_v[pl.ds(i, 16)] = jnp.full((16,), 1.0, jnp.float32)

    pltpu.sync_copy(zer_hbm.at[sl], acc.at[sl])
    pltpu.sync_copy(col_hbm.at[wid], col_v)
    plsc.subcore_barrier()

    @pl.loop(0, NCH, step=20)
    def _(j):
        for b in range(20):
            pltpu.async_copy(ones_v, acc.at[col_v.at[j + b]], sem, add=True)
        for b in range(20):
            pltpu.make_async_copy(ones_v, acc.at[col_v.at[j + b]], sem).wait()

    plsc.subcore_barrier()
    pltpu.sync_copy(acc.at[sl], out_hbm.at[c].at[sl])


def _scatter_body(D, g_hbm, zer_hbm, row_hbm, col_hbm, out_hbm,
                  row_v, col_v, bufs, g_st, acc, gsem, ssem):
    c = lax.axis_index("c")
    s = lax.axis_index("s")
    wid = s * NC + c
    sl = pl.ds(s * RPS, RPS)

    pltpu.sync_copy(g_hbm.at[sl], g_st.at[sl])

    @pl.when(c == 0)
    def _():
        pltpu.sync_copy(g_hbm.at[sl], acc.at[sl])

    @pl.when(c != 0)
    def _():
        pltpu.sync_copy(zer_hbm.at[sl], acc.at[sl])

    pltpu.sync_copy(row_hbm.at[wid], row_v)
    pltpu.sync_copy(col_hbm.at[wid], col_v)
    plsc.subcore_barrier()

    def start_gather(j, b):
        pltpu.async_copy(g_st.at[row_v.at[j]], bufs.at[b], gsem.at[b])

    def wait_gather(j, b):
        pltpu.make_async_copy(g_st.at[row_v.at[j]], bufs.at[b],
                              gsem.at[b]).wait()

    def start_scatter(j, b):
        pltpu.async_copy(bufs.at[b], acc.at[col_v.at[j]], ssem.at[b],
                         add=True)

    def wait_scatter(j, b):
        pltpu.make_async_copy(bufs.at[b], acc.at[col_v.at[j]],
                              ssem.at[b]).wait()

    for p in range(PF):
        start_gather(p, p)

    @pl.loop(0, NCH, step=NBUF)
    def _(j0):
        for i in range(NBUF):
            j = j0 + i
            bn = (i + PF) % NBUF
            jn = j + PF

            @pl.when(jn < NCH)
            def _():
                @pl.when(jn >= NBUF)
                def _():
                    wait_scatter(jn - NBUF, bn)
                start_gather(jn, bn)

            wait_gather(j, i)
            start_scatter(j, i)

    for b in range(NBUF):
        wait_scatter(0, b)

    plsc.subcore_barrier()
    pltpu.sync_copy(acc.at[sl], out_hbm.at[c].at[sl])


def _deg_call(colp, zer1):
    return pl.kernel(
        _deg_body,
        out_type=jax.ShapeDtypeStruct((NC, R_PAD), jnp.float32),
        mesh=_mesh,
        scratch_types=[
            pltpu.VMEM((NCH, CK), jnp.int32),
            pltpu.VMEM((CK,), jnp.float32),
            pltpu.VMEM_SHARED((R_PAD,), jnp.float32),
            pltpu.SemaphoreType.DMA,
        ],
        compiler_params=_sc_params,
    )(colp, zer1)


def _scatter_call(D, g, zer, rowp, colp):
    return pl.kernel(
        functools.partial(_scatter_body, D),
        out_type=jax.ShapeDtypeStruct((NC, R_PAD, D), jnp.float32),
        mesh=_mesh,
        scratch_types=[
            pltpu.VMEM((NCH, CK), jnp.int32),
            pltpu.VMEM((NCH, CK), jnp.int32),
            pltpu.VMEM((NBUF, CK, D), jnp.float32),
            pltpu.VMEM_SHARED((R_PAD, D), jnp.float32),
            pltpu.VMEM_SHARED((R_PAD, D), jnp.float32),
            pltpu.SemaphoreType.DMA((NBUF,)),
            pltpu.SemaphoreType.DMA((NBUF,)),
        ],
        compiler_params=_sc_params,
    )(g, zer, rowp, colp)


_BLK = 1024
_OBLK = 1000


def _mm1_body(x_ref, w_ref, o_ref):
    o_ref[...] = jnp.dot(x_ref[...], w_ref[...],
                         preferred_element_type=jnp.float32)


def _scale_body(p0_ref, p1_ref, h_ref, g_ref, d_ref):
    deg = p0_ref[...] + p1_ref[...] + 1.0
    d = lax.rsqrt(deg)
    d_ref[...] = d
    g_ref[...] = h_ref[...] * d


def _mid_body(a0_ref, a1_ref, d_ref, b1_ref, w2_ref, g2_ref):
    d = d_ref[...]
    o1 = jnp.maximum((a0_ref[...] + a1_ref[...]) * d + b1_ref[...], 0.0)
    h2 = jnp.dot(o1, w2_ref[...], preferred_element_type=jnp.float32)
    g2_ref[...] = h2 * d


def _fin_body(a0_ref, a1_ref, d_ref, b2_ref, o_ref):
    o = (a0_ref[...] + a1_ref[...]) * d_ref[...] + b2_ref[...]
    m = jnp.max(o, axis=1, keepdims=True)
    e = jnp.exp(o - m)
    lse = jnp.log(jnp.sum(e, axis=1, keepdims=True))
    o_ref[...] = o - m - lse


def _mm1(x_p, W1):
    return pl.pallas_call(
        _mm1_body,
        grid=(R_PAD // _BLK,),
        in_specs=[pl.BlockSpec((_BLK, F_IN), lambda i: (i, 0)),
                  pl.BlockSpec((F_IN, N_HID), lambda i: (0, 0))],
        out_specs=pl.BlockSpec((_BLK, N_HID), lambda i: (i, 0)),
        out_shape=jax.ShapeDtypeStruct((R_PAD, N_HID), jnp.float32),
    )(x_p, W1)


def _scale(p0, p1, h):
    return pl.pallas_call(
        _scale_body,
        grid=(R_PAD // _BLK,),
        in_specs=[pl.BlockSpec((_BLK, 1), lambda i: (i, 0)),
                  pl.BlockSpec((_BLK, 1), lambda i: (i, 0)),
                  pl.BlockSpec((_BLK, N_HID), lambda i: (i, 0))],
        out_specs=[pl.BlockSpec((_BLK, N_HID), lambda i: (i, 0)),
                   pl.BlockSpec((_BLK, 1), lambda i: (i, 0))],
        out_shape=[jax.ShapeDtypeStruct((R_PAD, N_HID), jnp.float32),
                   jax.ShapeDtypeStruct((R_PAD, 1), jnp.float32)],
    )(p0, p1, h)


def _mid(a0, a1, d, b1r, W2):
    return pl.pallas_call(
        _mid_body,
        grid=(R_PAD // _BLK,),
        in_specs=[pl.BlockSpec((_BLK, N_HID), lambda i: (i, 0)),
                  pl.BlockSpec((_BLK, N_HID), lambda i: (i, 0)),
                  pl.BlockSpec((_BLK, 1), lambda i: (i, 0)),
                  pl.BlockSpec((1, N_HID), lambda i: (0, 0)),
                  pl.BlockSpec((N_HID, N_CLS), lambda i: (0, 0))],
        out_specs=pl.BlockSpec((_BLK, N_CLS), lambda i: (i, 0)),
        out_shape=jax.ShapeDtypeStruct((R_PAD, N_CLS), jnp.float32),
    )(a0, a1, d, b1r, W2)


def _fin(a0, a1, d, b2r):
    return pl.pallas_call(
        _fin_body,
        grid=(N // _OBLK,),
        in_specs=[pl.BlockSpec((_OBLK, N_CLS), lambda i: (i, 0)),
                  pl.BlockSpec((_OBLK, N_CLS), lambda i: (i, 0)),
                  pl.BlockSpec((_OBLK, 1), lambda i: (i, 0)),
                  pl.BlockSpec((1, N_CLS), lambda i: (0, 0))],
        out_specs=pl.BlockSpec((_OBLK, N_CLS), lambda i: (i, 0)),
        out_shape=jax.ShapeDtypeStruct((N, N_CLS), jnp.float32),
    )(a0, a1, d, b2r)


def kernel(x, edge_index, W1, b1, W2, b2):
    row = edge_index[0]
    col = edge_index[1]
    pad = E_PAD - E
    rowp = jnp.concatenate(
        [row, jnp.zeros((pad,), jnp.int32)]).reshape(NW, NCH, CK)
    colp = jnp.concatenate(
        [col, jnp.full((pad,), N, jnp.int32)]).reshape(NW, NCH, CK)
    x_p = jnp.pad(x, ((0, R_PAD - N), (0, 0)))

    zer1 = jnp.zeros((R_PAD,), jnp.float32)
    zer16 = jnp.zeros((R_PAD, N_HID), jnp.float32)
    zer40 = jnp.zeros((R_PAD, N_CLS), jnp.float32)

    degp = _deg_call(colp, zer1)
    h = _mm1(x_p, W1)
    g1, d = _scale(degp[0].reshape(R_PAD, 1), degp[1].reshape(R_PAD, 1), h)
    a1 = _scatter_call(N_HID, g1, zer16, rowp, colp)
    g2 = _mid(a1[0], a1[1], d, b1.reshape(1, N_HID), W2)
    a2 = _scatter_call(N_CLS, g2, zer40, rowp, colp)
    return _fin(a2[0], a2[1], d, b2.reshape(1, N_CLS))


# speedup vs baseline: 45.3838x; 1.0019x over previous
import functools

import jax
import jax.numpy as jnp
from jax import lax
from jax.experimental import pallas as pl
from jax.experimental.pallas import tpu as pltpu
from jax.experimental.pallas import tpu_sc as plsc

N = 10000
E = 320000
F_IN = 128
N_HID = 16
N_CLS = 40

NC = 2
NS = 16
NW = NC * NS
CK = 128
NCH = 80
E_PAD = NW * NCH * CK
R_PAD = 10240
RPS = R_PAD // NS
NBUF = 10
PF = 5

_mesh = plsc.VectorSubcoreMesh(core_axis_name="c", subcore_axis_name="s")
_sc_params = pltpu.CompilerParams(use_tc_tiling_on_sc=False)


def _deg_body(col_hbm, zer_hbm, out_hbm, col_v, ones_v, acc, sem):
    c = lax.axis_index("c")
    s = lax.axis_index("s")
    wid = s * NC + c
    sl = pl.ds(s * RPS, RPS)

    @pl.loop(0, CK, step=16)
    def _(i):
        ones_v[pl.ds(i, 16)] = jnp.full((16,), 1.0, jnp.float32)

    pltpu.sync_copy(zer_hbm.at[sl], acc.at[sl])
    pltpu.sync_copy(col_hbm.at[wid], col_v)
    plsc.subcore_barrier()

    @pl.loop(0, NCH, step=20)
    def _(j):
        for b in range(20):
            pltpu.async_copy(ones_v, acc.at[col_v.at[j + b]], sem, add=True)
        for b in range(20):
            pltpu.make_async_copy(ones_v, acc.at[col_v.at[j + b]], sem).wait()

    plsc.subcore_barrier()
    pltpu.sync_copy(acc.at[sl], out_hbm.at[c].at[sl])


def _scatter_body(D, g_hbm, zer_hbm, row_hbm, col_hbm, out_hbm,
                  row_v, col_v, bufs, g_st, acc, gsem, ssem):
    c = lax.axis_index("c")
    s = lax.axis_index("s")
    wid = s * NC + c
    sl = pl.ds(s * RPS, RPS)

    pltpu.sync_copy(g_hbm.at[sl], g_st.at[sl])

    @pl.when(c == 0)
    def _():
        pltpu.sync_copy(g_hbm.at[sl], acc.at[sl])

    @pl.when(c != 0)
    def _():
        pltpu.sync_copy(zer_hbm.at[sl], acc.at[sl])

    pltpu.sync_copy(row_hbm.at[wid], row_v)
    pltpu.sync_copy(col_hbm.at[wid], col_v)
    plsc.subcore_barrier()

    def start_gather(j, b):
        pltpu.async_copy(g_st.at[row_v.at[j]], bufs.at[b], gsem.at[b])

    def wait_gather(j, b):
        pltpu.make_async_copy(g_st.at[row_v.at[j]], bufs.at[b],
                              gsem.at[b]).wait()

    def start_scatter(j, b):
        pltpu.async_copy(bufs.at[b], acc.at[col_v.at[j]], ssem.at[b],
                         add=True)

    def wait_scatter(j, b):
        pltpu.make_async_copy(bufs.at[b], acc.at[col_v.at[j]],
                              ssem.at[b]).wait()

    for p in range(PF):
        start_gather(p, p)

    @pl.loop(0, NCH, step=NBUF)
    def _(j0):
        for i in range(NBUF):
            j = j0 + i
            bn = (i + PF) % NBUF
            jn = j + PF

            @pl.when(jn < NCH)
            def _():
                @pl.when(jn >= NBUF)
                def _():
                    wait_scatter(jn - NBUF, bn)
                start_gather(jn, bn)

            wait_gather(j, i)
            start_scatter(j, i)

    for b in range(NBUF):
        wait_scatter(0, b)

    plsc.subcore_barrier()
    pltpu.sync_copy(acc.at[sl], out_hbm.at[c].at[sl])


def _deg_call(colp, zer1):
    return pl.kernel(
        _deg_body,
        out_type=jax.ShapeDtypeStruct((NC, R_PAD), jnp.float32),
        mesh=_mesh,
        scratch_types=[
            pltpu.VMEM((NCH, CK), jnp.int32),
            pltpu.VMEM((CK,), jnp.float32),
            pltpu.VMEM_SHARED((R_PAD,), jnp.float32),
            pltpu.SemaphoreType.DMA,
        ],
        compiler_params=_sc_params,
    )(colp, zer1)


def _scatter_call(D, g, zer, rowp, colp):
    return pl.kernel(
        functools.partial(_scatter_body, D),
        out_type=jax.ShapeDtypeStruct((NC, R_PAD, D), jnp.float32),
        mesh=_mesh,
        scratch_types=[
            pltpu.VMEM((NCH, CK), jnp.int32),
            pltpu.VMEM((NCH, CK), jnp.int32),
            pltpu.VMEM((NBUF, CK, D), jnp.float32),
            pltpu.VMEM_SHARED((R_PAD, D), jnp.float32),
            pltpu.VMEM_SHARED((R_PAD, D), jnp.float32),
            pltpu.SemaphoreType.DMA((NBUF,)),
            pltpu.SemaphoreType.DMA((NBUF,)),
        ],
        compiler_params=_sc_params,
    )(g, zer, rowp, colp)


_BLK = 1024
_OBLK = 1000


def _scale_body(p0_ref, p1_ref, x_ref, w_ref, g_ref, d_ref):
    deg = p0_ref[...] + p1_ref[...] + 1.0
    d = lax.rsqrt(deg)
    d_ref[...] = d
    h = jnp.dot(x_ref[...], w_ref[...], preferred_element_type=jnp.float32)
    g_ref[...] = h * d


def _mid_body(a0_ref, a1_ref, d_ref, b1_ref, w2_ref, g2_ref):
    d = d_ref[...]
    o1 = jnp.maximum((a0_ref[...] + a1_ref[...]) * d + b1_ref[...], 0.0)
    h2 = jnp.dot(o1, w2_ref[...], preferred_element_type=jnp.float32)
    g2_ref[...] = h2 * d


def _fin_body(a0_ref, a1_ref, d_ref, b2_ref, o_ref):
    o = (a0_ref[...] + a1_ref[...]) * d_ref[...] + b2_ref[...]
    m = jnp.max(o, axis=1, keepdims=True)
    e = jnp.exp(o - m)
    lse = jnp.log(jnp.sum(e, axis=1, keepdims=True))
    o_ref[...] = o - m - lse


def _scale(p0, p1, x_p, W1):
    return pl.pallas_call(
        _scale_body,
        grid=(R_PAD // _BLK,),
        in_specs=[pl.BlockSpec((_BLK, 1), lambda i: (i, 0)),
                  pl.BlockSpec((_BLK, 1), lambda i: (i, 0)),
                  pl.BlockSpec((_BLK, F_IN), lambda i: (i, 0)),
                  pl.BlockSpec((F_IN, N_HID), lambda i: (0, 0))],
        out_specs=[pl.BlockSpec((_BLK, N_HID), lambda i: (i, 0)),
                   pl.BlockSpec((_BLK, 1), lambda i: (i, 0))],
        out_shape=[jax.ShapeDtypeStruct((R_PAD, N_HID), jnp.float32),
                   jax.ShapeDtypeStruct((R_PAD, 1), jnp.float32)],
    )(p0, p1, x_p, W1)


def _mid(a0, a1, d, b1r, W2):
    return pl.pallas_call(
        _mid_body,
        grid=(R_PAD // _BLK,),
        in_specs=[pl.BlockSpec((_BLK, N_HID), lambda i: (i, 0)),
                  pl.BlockSpec((_BLK, N_HID), lambda i: (i, 0)),
                  pl.BlockSpec((_BLK, 1), lambda i: (i, 0)),
                  pl.BlockSpec((1, N_HID), lambda i: (0, 0)),
                  pl.BlockSpec((N_HID, N_CLS), lambda i: (0, 0))],
        out_specs=pl.BlockSpec((_BLK, N_CLS), lambda i: (i, 0)),
        out_shape=jax.ShapeDtypeStruct((R_PAD, N_CLS), jnp.float32),
    )(a0, a1, d, b1r, W2)


def _fin(a0, a1, d, b2r):
    return pl.pallas_call(
        _fin_body,
        grid=(N // _OBLK,),
        in_specs=[pl.BlockSpec((_OBLK, N_CLS), lambda i: (i, 0)),
                  pl.BlockSpec((_OBLK, N_CLS), lambda i: (i, 0)),
                  pl.BlockSpec((_OBLK, 1), lambda i: (i, 0)),
                  pl.BlockSpec((1, N_CLS), lambda i: (0, 0))],
        out_specs=pl.BlockSpec((_OBLK, N_CLS), lambda i: (i, 0)),
        out_shape=jax.ShapeDtypeStruct((N, N_CLS), jnp.float32),
    )(a0, a1, d, b2r)


def kernel(x, edge_index, W1, b1, W2, b2):
    row = edge_index[0]
    col = edge_index[1]
    pad = E_PAD - E
    rowp = jnp.concatenate(
        [row, jnp.zeros((pad,), jnp.int32)]).reshape(NW, NCH, CK)
    colp = jnp.concatenate(
        [col, jnp.full((pad,), N, jnp.int32)]).reshape(NW, NCH, CK)
    x_p = jnp.pad(x, ((0, R_PAD - N), (0, 0)))

    zer1 = jnp.zeros((R_PAD,), jnp.float32)
    zer16 = jnp.zeros((R_PAD, N_HID), jnp.float32)
    zer40 = jnp.zeros((R_PAD, N_CLS), jnp.float32)

    degp = _deg_call(colp, zer1)
    g1, d = _scale(degp[0].reshape(R_PAD, 1), degp[1].reshape(R_PAD, 1),
                   x_p, W1)
    a1 = _scatter_call(N_HID, g1, zer16, rowp, colp)
    g2 = _mid(a1[0], a1[1], d, b1.reshape(1, N_HID), W2)
    a2 = _scatter_call(N_CLS, g2, zer40, rowp, colp)
    return _fin(a2[0], a2[1], d, b2.reshape(1, N_CLS))


# speedup vs baseline: 46.5137x; 1.0249x over previous
import functools

import jax
import jax.numpy as jnp
from jax import lax
from jax.experimental import pallas as pl
from jax.experimental.pallas import tpu as pltpu
from jax.experimental.pallas import tpu_sc as plsc

N = 10000
E = 320000
F_IN = 128
N_HID = 16
N_CLS = 40

NC = 2
NS = 16
NW = NC * NS
CK = 128
NCH = 80
E_PAD = NW * NCH * CK
R_PAD = 10240
RPS = R_PAD // NS
NBUF = 10
PF = 5

_mesh = plsc.VectorSubcoreMesh(core_axis_name="c", subcore_axis_name="s")
_sc_params = pltpu.CompilerParams(use_tc_tiling_on_sc=False)


def _deg_body(col_hbm, zer_hbm, out_hbm, col_v, ones_v, acc, sem):
    c = lax.axis_index("c")
    s = lax.axis_index("s")
    wid = s * NC + c
    sl = pl.ds(s * RPS, RPS)

    pltpu.async_copy(zer_hbm.at[sl], acc.at[sl], sem)
    pltpu.async_copy(col_hbm.at[wid], col_v, sem)

    @pl.loop(0, CK, step=16)
    def _(i):
        ones_v[pl.ds(i, 16)] = jnp.full((16,), 1.0, jnp.float32)

    pltpu.make_async_copy(zer_hbm.at[sl], acc.at[sl], sem).wait()
    pltpu.make_async_copy(col_hbm.at[wid], col_v, sem).wait()
    plsc.subcore_barrier()

    @pl.loop(0, NCH, step=20)
    def _(j):
        for b in range(20):
            pltpu.async_copy(ones_v, acc.at[col_v.at[j + b]], sem, add=True)
        for b in range(20):
            pltpu.make_async_copy(ones_v, acc.at[col_v.at[j + b]], sem).wait()

    plsc.subcore_barrier()
    pltpu.sync_copy(acc.at[sl], out_hbm.at[c].at[sl])


def _scatter_body(D, g_hbm, zer_hbm, row_hbm, col_hbm, out_hbm,
                  row_v, col_v, bufs, g_st, acc, gsem, ssem):
    c = lax.axis_index("c")
    s = lax.axis_index("s")
    wid = s * NC + c
    sl = pl.ds(s * RPS, RPS)

    pltpu.async_copy(g_hbm.at[sl], g_st.at[sl], ssem.at[0])

    @pl.when(c == 0)
    def _():
        pltpu.async_copy(g_hbm.at[sl], acc.at[sl], ssem.at[1])

    @pl.when(c != 0)
    def _():
        pltpu.async_copy(zer_hbm.at[sl], acc.at[sl], ssem.at[1])

    pltpu.async_copy(row_hbm.at[wid], row_v, ssem.at[2])
    pltpu.async_copy(col_hbm.at[wid], col_v, ssem.at[3])

    pltpu.make_async_copy(g_hbm.at[sl], g_st.at[sl], ssem.at[0]).wait()
    pltpu.make_async_copy(zer_hbm.at[sl], acc.at[sl], ssem.at[1]).wait()
    pltpu.make_async_copy(row_hbm.at[wid], row_v, ssem.at[2]).wait()
    pltpu.make_async_copy(col_hbm.at[wid], col_v, ssem.at[3]).wait()
    plsc.subcore_barrier()

    def start_gather(j, b):
        pltpu.async_copy(g_st.at[row_v.at[j]], bufs.at[b], gsem.at[b])

    def wait_gather(j, b):
        pltpu.make_async_copy(g_st.at[row_v.at[j]], bufs.at[b],
                              gsem.at[b]).wait()

    def start_scatter(j, b):
        pltpu.async_copy(bufs.at[b], acc.at[col_v.at[j]], ssem.at[b],
                         add=True)

    def wait_scatter(j, b):
        pltpu.make_async_copy(bufs.at[b], acc.at[col_v.at[j]],
                              ssem.at[b]).wait()

    for p in range(PF):
        start_gather(p, p)

    @pl.loop(0, NCH, step=NBUF)
    def _(j0):
        for i in range(NBUF):
            j = j0 + i
            bn = (i + PF) % NBUF
            jn = j + PF

            @pl.when(jn < NCH)
            def _():
                @pl.when(jn >= NBUF)
                def _():
                    wait_scatter(jn - NBUF, bn)
                start_gather(jn, bn)

            wait_gather(j, i)
            start_scatter(j, i)

    for b in range(NBUF):
        wait_scatter(0, b)

    plsc.subcore_barrier()
    pltpu.sync_copy(acc.at[sl], out_hbm.at[c].at[sl])


def _deg_call(colp, zer1):
    return pl.kernel(
        _deg_body,
        out_type=jax.ShapeDtypeStruct((NC, R_PAD), jnp.float32),
        mesh=_mesh,
        scratch_types=[
            pltpu.VMEM((NCH, CK), jnp.int32),
            pltpu.VMEM((CK,), jnp.float32),
            pltpu.VMEM_SHARED((R_PAD,), jnp.float32),
            pltpu.SemaphoreType.DMA,
        ],
        compiler_params=_sc_params,
    )(colp, zer1)


def _scatter_call(D, g, zer, rowp, colp):
    return pl.kernel(
        functools.partial(_scatter_body, D),
        out_type=jax.ShapeDtypeStruct((NC, R_PAD, D), jnp.float32),
        mesh=_mesh,
        scratch_types=[
            pltpu.VMEM((NCH, CK), jnp.int32),
            pltpu.VMEM((NCH, CK), jnp.int32),
            pltpu.VMEM((NBUF, CK, D), jnp.float32),
            pltpu.VMEM_SHARED((R_PAD, D), jnp.float32),
            pltpu.VMEM_SHARED((R_PAD, D), jnp.float32),
            pltpu.SemaphoreType.DMA((NBUF,)),
            pltpu.SemaphoreType.DMA((NBUF,)),
        ],
        compiler_params=_sc_params,
    )(g, zer, rowp, colp)


_BLK = 1024
_OBLK = 1000


def _scale_body(p0_ref, p1_ref, x_ref, w_ref, g_ref, d_ref):
    deg = p0_ref[...] + p1_ref[...] + 1.0
    d = lax.rsqrt(deg)
    d_ref[...] = d
    h = jnp.dot(x_ref[...], w_ref[...], preferred_element_type=jnp.float32)
    g_ref[...] = h * d


def _mid_body(a0_ref, a1_ref, d_ref, b1_ref, w2_ref, g2_ref):
    d = d_ref[...]
    o1 = jnp.maximum((a0_ref[...] + a1_ref[...]) * d + b1_ref[...], 0.0)
    h2 = jnp.dot(o1, w2_ref[...], preferred_element_type=jnp.float32)
    g2_ref[...] = h2 * d


def _fin_body(a0_ref, a1_ref, d_ref, b2_ref, o_ref):
    o = (a0_ref[...] + a1_ref[...]) * d_ref[...] + b2_ref[...]
    m = jnp.max(o, axis=1, keepdims=True)
    e = jnp.exp(o - m)
    lse = jnp.log(jnp.sum(e, axis=1, keepdims=True))
    o_ref[...] = o - m - lse


def _scale(p0, p1, x_p, W1):
    return pl.pallas_call(
        _scale_body,
        grid=(R_PAD // _BLK,),
        in_specs=[pl.BlockSpec((_BLK, 1), lambda i: (i, 0)),
                  pl.BlockSpec((_BLK, 1), lambda i: (i, 0)),
                  pl.BlockSpec((_BLK, F_IN), lambda i: (i, 0)),
                  pl.BlockSpec((F_IN, N_HID), lambda i: (0, 0))],
        out_specs=[pl.BlockSpec((_BLK, N_HID), lambda i: (i, 0)),
                   pl.BlockSpec((_BLK, 1), lambda i: (i, 0))],
        out_shape=[jax.ShapeDtypeStruct((R_PAD, N_HID), jnp.float32),
                   jax.ShapeDtypeStruct((R_PAD, 1), jnp.float32)],
    )(p0, p1, x_p, W1)


def _mid(a0, a1, d, b1r, W2):
    return pl.pallas_call(
        _mid_body,
        grid=(R_PAD // _BLK,),
        in_specs=[pl.BlockSpec((_BLK, N_HID), lambda i: (i, 0)),
                  pl.BlockSpec((_BLK, N_HID), lambda i: (i, 0)),
                  pl.BlockSpec((_BLK, 1), lambda i: (i, 0)),
                  pl.BlockSpec((1, N_HID), lambda i: (0, 0)),
                  pl.BlockSpec((N_HID, N_CLS), lambda i: (0, 0))],
        out_specs=pl.BlockSpec((_BLK, N_CLS), lambda i: (i, 0)),
        out_shape=jax.ShapeDtypeStruct((R_PAD, N_CLS), jnp.float32),
    )(a0, a1, d, b1r, W2)


def _fin(a0, a1, d, b2r):
    return pl.pallas_call(
        _fin_body,
        grid=(N // _OBLK,),
        in_specs=[pl.BlockSpec((_OBLK, N_CLS), lambda i: (i, 0)),
                  pl.BlockSpec((_OBLK, N_CLS), lambda i: (i, 0)),
                  pl.BlockSpec((_OBLK, 1), lambda i: (i, 0)),
                  pl.BlockSpec((1, N_CLS), lambda i: (0, 0))],
        out_specs=pl.BlockSpec((_OBLK, N_CLS), lambda i: (i, 0)),
        out_shape=jax.ShapeDtypeStruct((N, N_CLS), jnp.float32),
    )(a0, a1, d, b2r)


def kernel(x, edge_index, W1, b1, W2, b2):
    row = edge_index[0]
    col = edge_index[1]
    pad = E_PAD - E
    rowp = jnp.concatenate(
        [row, jnp.zeros((pad,), jnp.int32)]).reshape(NW, NCH, CK)
    colp = jnp.concatenate(
        [col, jnp.full((pad,), N, jnp.int32)]).reshape(NW, NCH, CK)
    x_p = jnp.pad(x, ((0, R_PAD - N), (0, 0)))

    zer1 = jnp.zeros((R_PAD,), jnp.float32)
    zer16 = jnp.zeros((R_PAD, N_HID), jnp.float32)
    zer40 = jnp.zeros((R_PAD, N_CLS), jnp.float32)

    degp = _deg_call(colp, zer1)
    g1, d = _scale(degp[0].reshape(R_PAD, 1), degp[1].reshape(R_PAD, 1),
                   x_p, W1)
    a1 = _scatter_call(N_HID, g1, zer16, rowp, colp)
    g2 = _mid(a1[0], a1[1], d, b1.reshape(1, N_HID), W2)
    a2 = _scatter_call(N_CLS, g2, zer40, rowp, colp)
    return _fin(a2[0], a2[1], d, b2.reshape(1, N_CLS))


# speedup vs baseline: 57.3277x; 1.2325x over previous
import functools

import jax
import jax.numpy as jnp
from jax import lax
from jax.experimental import pallas as pl
from jax.experimental.pallas import tpu as pltpu
from jax.experimental.pallas import tpu_sc as plsc

N = 10000
E = 320000
F_IN = 128
N_HID = 16
N_CLS = 40

NC = 2
NS = 16
NW = NC * NS
CK = 128
CTOT = E // CK
NCHW = CTOT // NW
TAIL_BASE = NCHW * NW
N_TAIL = CTOT - TAIL_BASE
R_PAD = 10240
RPS = R_PAD // NS
NBUF = 8
PF = 5
NCH_UP = ((NCHW + NBUF - 1) // NBUF) * NBUF

_mesh = plsc.VectorSubcoreMesh(core_axis_name="c", subcore_axis_name="s")
_sc_params = pltpu.CompilerParams(use_tc_tiling_on_sc=False)


def _deg_body(e4_hbm, zer_hbm, one_hbm, out_hbm, col_v, ones_v, colx, acc,
              sem):
    c = lax.axis_index("c")
    s = lax.axis_index("s")
    wid = s * NC + c
    base = wid * NCHW
    sl = pl.ds(s * RPS, RPS)

    pltpu.async_copy(zer_hbm.at[sl], acc.at[sl], sem)
    pltpu.async_copy(e4_hbm.at[1, pl.ds(base, NCHW)], col_v, sem)
    pltpu.async_copy(one_hbm, ones_v, sem)
    pltpu.make_async_copy(zer_hbm.at[sl], acc.at[sl], sem).wait()
    pltpu.make_async_copy(e4_hbm.at[1, pl.ds(base, NCHW)], col_v, sem).wait()
    pltpu.make_async_copy(one_hbm, ones_v, sem).wait()
    plsc.subcore_barrier()

    @pl.loop(0, NCHW, step=13)
    def _(j):
        for b in range(13):
            pltpu.async_copy(ones_v, acc.at[col_v.at[j + b]], sem, add=True)
        for b in range(13):
            pltpu.make_async_copy(ones_v, acc.at[col_v.at[j + b]], sem).wait()

    @pl.when(wid < N_TAIL)
    def _():
        pltpu.sync_copy(e4_hbm.at[1, TAIL_BASE + wid], colx)
        pltpu.sync_copy(ones_v, acc.at[colx], add=True)

    plsc.subcore_barrier()
    pltpu.sync_copy(acc.at[sl], out_hbm.at[c].at[sl])


def _scatter_body(D, g_hbm, zer_hbm, e4_hbm, out_hbm,
                  idx_v, idxx, bufs, g_st, acc, gsem, ssem):
    c = lax.axis_index("c")
    s = lax.axis_index("s")
    wid = s * NC + c
    base = wid * NCHW
    sl = pl.ds(s * RPS, RPS)

    pltpu.async_copy(g_hbm.at[sl], g_st.at[sl], ssem.at[0])

    @pl.when(c == 0)
    def _():
        pltpu.async_copy(g_hbm.at[sl], acc.at[sl], ssem.at[1])

    @pl.when(c != 0)
    def _():
        pltpu.async_copy(zer_hbm.at[sl], acc.at[sl], ssem.at[1])

    pltpu.async_copy(e4_hbm.at[:, pl.ds(base, NCHW)], idx_v, ssem.at[2])

    pltpu.make_async_copy(g_hbm.at[sl], g_st.at[sl], ssem.at[0]).wait()
    pltpu.make_async_copy(zer_hbm.at[sl], acc.at[sl], ssem.at[1]).wait()
    pltpu.make_async_copy(e4_hbm.at[:, pl.ds(base, NCHW)], idx_v,
                          ssem.at[2]).wait()
    plsc.subcore_barrier()

    def start_gather(j, b):
        pltpu.async_copy(g_st.at[idx_v.at[0, j]], bufs.at[b], gsem.at[b])

    def wait_gather(j, b):
        pltpu.make_async_copy(g_st.at[idx_v.at[0, j]], bufs.at[b],
                              gsem.at[b]).wait()

    def start_scatter(j, b):
        pltpu.async_copy(bufs.at[b], acc.at[idx_v.at[1, j]], ssem.at[b],
                         add=True)

    def wait_scatter(j, b):
        pltpu.make_async_copy(bufs.at[b], acc.at[idx_v.at[1, j]],
                              ssem.at[b]).wait()

    for p in range(PF):
        start_gather(p, p)

    @pl.loop(0, NCH_UP, step=NBUF)
    def _(j0):
        for i in range(NBUF):
            j = j0 + i
            bn = (i + PF) % NBUF
            jn = j + PF

            @pl.when(jn < NCHW)
            def _():
                @pl.when(jn >= NBUF)
                def _():
                    wait_scatter(jn - NBUF, bn)
                start_gather(jn, bn)

            @pl.when(j < NCHW)
            def _():
                wait_gather(j, i)
                start_scatter(j, i)

    for b in range(NBUF):
        wait_scatter(0, b)

    @pl.when(wid < N_TAIL)
    def _():
        pltpu.sync_copy(e4_hbm.at[:, TAIL_BASE + wid], idxx)
        pltpu.sync_copy(g_st.at[idxx.at[0]], bufs.at[0])
        pltpu.sync_copy(bufs.at[0], acc.at[idxx.at[1]], add=True)

    plsc.subcore_barrier()
    pltpu.sync_copy(acc.at[sl], out_hbm.at[c].at[sl])


def _deg_call(e4, zer1, one1):
    return pl.kernel(
        _deg_body,
        out_type=jax.ShapeDtypeStruct((NC, R_PAD, 1), jnp.float32),
        mesh=_mesh,
        scratch_types=[
            pltpu.VMEM((NCHW, CK), jnp.int32),
            pltpu.VMEM((CK, 1), jnp.float32),
            pltpu.VMEM((CK,), jnp.int32),
            pltpu.VMEM_SHARED((R_PAD, 1), jnp.float32),
            pltpu.SemaphoreType.DMA,
        ],
        compiler_params=_sc_params,
    )(e4, zer1, one1)


def _scatter_call(D, g, zer, e4):
    return pl.kernel(
        functools.partial(_scatter_body, D),
        out_type=jax.ShapeDtypeStruct((NC, R_PAD, D), jnp.float32),
        mesh=_mesh,
        scratch_types=[
            pltpu.VMEM((2, NCHW, CK), jnp.int32),
            pltpu.VMEM((2, CK), jnp.int32),
            pltpu.VMEM((NBUF, CK, D), jnp.float32),
            pltpu.VMEM_SHARED((R_PAD, D), jnp.float32),
            pltpu.VMEM_SHARED((R_PAD, D), jnp.float32),
            pltpu.SemaphoreType.DMA((NBUF,)),
            pltpu.SemaphoreType.DMA((NBUF,)),
        ],
        compiler_params=_sc_params,
    )(g, zer, e4)


def _scale_body(degp_ref, x_ref, w_ref, g_ref, d_ref):
    degp = degp_ref[...]
    deg = degp[0] + degp[1] + 1.0
    d = lax.rsqrt(deg)
    d_ref[...] = d
    h = jnp.dot(x_ref[...], w_ref[...], preferred_element_type=jnp.float32)
    g_ref[...] = h * d[:N, :]


def _mid_body(a_ref, d_ref, b1_ref, w2_ref, g2_ref):
    a = a_ref[...]
    d = d_ref[...]
    o1 = jnp.maximum((a[0] + a[1]) * d + b1_ref[...], 0.0)
    h2 = jnp.dot(o1, w2_ref[...], preferred_element_type=jnp.float32)
    g2_ref[...] = h2 * d


def _fin_body(a_ref, d_ref, b2_ref, o_ref):
    a = a_ref[...]
    o = (a[0, :N] + a[1, :N]) * d_ref[...][:N] + b2_ref[...]
    m = jnp.max(o, axis=1, keepdims=True)
    e = jnp.exp(o - m)
    lse = jnp.log(jnp.sum(e, axis=1, keepdims=True))
    o_ref[...] = o - m - lse


def _scale(degp, x, W1):
    return pl.pallas_call(
        _scale_body,
        grid=(1,),
        in_specs=[pl.BlockSpec((NC, R_PAD, 1), lambda i: (0, 0, 0)),
                  pl.BlockSpec((N, F_IN), lambda i: (0, 0)),
                  pl.BlockSpec((F_IN, N_HID), lambda i: (0, 0))],
        out_specs=[pl.BlockSpec((N, N_HID), lambda i: (0, 0)),
                   pl.BlockSpec((R_PAD, 1), lambda i: (0, 0))],
        out_shape=[jax.ShapeDtypeStruct((R_PAD, N_HID), jnp.float32),
                   jax.ShapeDtypeStruct((R_PAD, 1), jnp.float32)],
    )(degp, x, W1)


def _mid(a1, d, b1r, W2):
    return pl.pallas_call(
        _mid_body,
        grid=(1,),
        in_specs=[pl.BlockSpec((NC, R_PAD, N_HID), lambda i: (0, 0, 0)),
                  pl.BlockSpec((R_PAD, 1), lambda i: (0, 0)),
                  pl.BlockSpec((1, N_HID), lambda i: (0, 0)),
                  pl.BlockSpec((N_HID, N_CLS), lambda i: (0, 0))],
        out_specs=pl.BlockSpec((R_PAD, N_CLS), lambda i: (0, 0)),
        out_shape=jax.ShapeDtypeStruct((R_PAD, N_CLS), jnp.float32),
    )(a1, d, b1r, W2)


def _fin(a2, d, b2r):
    return pl.pallas_call(
        _fin_body,
        grid=(1,),
        in_specs=[pl.BlockSpec((NC, R_PAD, N_CLS), lambda i: (0, 0, 0)),
                  pl.BlockSpec((R_PAD, 1), lambda i: (0, 0)),
                  pl.BlockSpec((1, N_CLS), lambda i: (0, 0))],
        out_specs=pl.BlockSpec((N, N_CLS), lambda i: (0, 0)),
        out_shape=jax.ShapeDtypeStruct((N, N_CLS), jnp.float32),
    )(a2, d, b2r)


def kernel(x, edge_index, W1, b1, W2, b2):
    e4 = edge_index.reshape(2, CTOT, CK)

    zer1 = jnp.zeros((R_PAD, 1), jnp.float32)
    one1 = jnp.ones((CK, 1), jnp.float32)
    zer16 = jnp.zeros((R_PAD, N_HID), jnp.float32)
    zer40 = jnp.zeros((R_PAD, N_CLS), jnp.float32)

    degp = _deg_call(e4, zer1, one1)
    g1, d = _scale(degp, x, W1)
    a1 = _scatter_call(N_HID, g1, zer16, e4)
    g2 = _mid(a1, d, b1.reshape(1, N_HID), W2)
    a2 = _scatter_call(N_CLS, g2, zer40, e4)
    return _fin(a2, d, b2.reshape(1, N_CLS))
